# baseline (speedup 1.0000x reference)
import functools

import jax
import jax.numpy as jnp
from jax.experimental import pallas as pl
from jax.experimental.pallas import tpu as pltpu

BT = 512


def _mix_layer(xb, ws, bs, wf, bp):
    P = ws.shape[-1]
    logits = jnp.dot(xb, ws, preferred_element_type=jnp.float32) + bs
    m = jnp.max(logits, axis=-1, keepdims=True)
    e = jnp.exp(logits - m)
    probs = e / jnp.sum(e, axis=-1, keepdims=True)
    pb = probs.astype(jnp.bfloat16)
    xp = jnp.concatenate([xb * pb[:, p : p + 1] for p in range(P)], axis=1)
    acc = jnp.dot(xp, wf, preferred_element_type=jnp.float32)
    return acc + jnp.dot(probs, bp, preferred_element_type=jnp.float32)


def _drn_body(x_ref, ws0, bs0, wf0, bp0, ws1, bs1, wf1, bp1, wc, bc, o_ref):
    xb = x_ref[...]
    h1 = _mix_layer(xb, ws0[...], bs0[...], wf0[...], bp0[...])
    h1 = jnp.maximum(h1, 0.0).astype(jnp.bfloat16)
    h2 = _mix_layer(h1, ws1[...], bs1[...], wf1[...], bp1[...])
    h2 = jnp.maximum(h2, 0.0).astype(jnp.bfloat16)
    o_ref[...] = jnp.dot(h2, wc[...], preferred_element_type=jnp.float32) + bc[...]


@jax.jit
def kernel(x, W_sel0, b_sel0, W_pops0, b_pops0, W_sel1, b_sel1, W_pops1,
           b_pops1, W_cls, b_cls):
    T, D = x.shape
    P, _, H1 = W_pops0.shape
    H2 = W_pops1.shape[-1]
    OUT = W_cls.shape[-1]
    bf16 = jnp.bfloat16

    args = (
        x.astype(bf16),
        W_sel0.astype(bf16), b_sel0.reshape(1, P),
        W_pops0.astype(bf16).reshape(P * D, H1), b_pops0,
        W_sel1.astype(bf16), b_sel1.reshape(1, P),
        W_pops1.astype(bf16).reshape(P * H1, H2), b_pops1,
        W_cls.astype(bf16), b_cls.reshape(1, OUT),
    )

    def const(shape):
        return pl.BlockSpec(shape, lambda i: (0,) * len(shape))

    return pl.pallas_call(
        _drn_body,
        grid=(T // BT,),
        in_specs=[
            pl.BlockSpec((BT, D), lambda i: (i, 0)),
            const((D, P)), const((1, P)),
            const((P * D, H1)), const((P, H1)),
            const((H1, P)), const((1, P)),
            const((P * H1, H2)), const((P, H2)),
            const((H2, OUT)), const((1, OUT)),
        ],
        out_specs=pl.BlockSpec((BT, OUT), lambda i: (i, 0)),
        out_shape=jax.ShapeDtypeStruct((T, OUT), jnp.float32),
        compiler_params=pltpu.CompilerParams(
            dimension_semantics=("parallel",),
        ),
    )(*args)


# speedup vs baseline: 1.0338x; 1.0338x over previous
import functools

import jax
import jax.numpy as jnp
from jax.experimental import pallas as pl
from jax.experimental.pallas import tpu as pltpu

BT = 512


def _mix_layer(xb, ws, bs, wp, bp):
    P = ws.shape[-1]
    logits = jnp.dot(xb, ws, preferred_element_type=jnp.float32) + bs
    m = jnp.max(logits, axis=-1, keepdims=True)
    e = jnp.exp(logits - m)
    probs = e / jnp.sum(e, axis=-1, keepdims=True)
    acc = jnp.dot(probs, bp, preferred_element_type=jnp.float32)
    for p in range(P):
        y = jnp.dot(xb, wp[p], preferred_element_type=jnp.float32)
        acc += probs[:, p : p + 1] * y
    return acc


def _drn_body(x_ref, ws0, bs0, wp0, bp0, ws1, bs1, wp1, bp1, wc, bc, o_ref):
    xb = x_ref[...]
    h1 = _mix_layer(xb, ws0[...], bs0[...], wp0, bp0[...])
    h1 = jnp.maximum(h1, 0.0).astype(jnp.bfloat16)
    h2 = _mix_layer(h1, ws1[...], bs1[...], wp1, bp1[...])
    h2 = jnp.maximum(h2, 0.0).astype(jnp.bfloat16)
    o_ref[...] = jnp.dot(h2, wc[...], preferred_element_type=jnp.float32) + bc[...]


@jax.jit
def kernel(x, W_sel0, b_sel0, W_pops0, b_pops0, W_sel1, b_sel1, W_pops1,
           b_pops1, W_cls, b_cls):
    T, D = x.shape
    P, _, H1 = W_pops0.shape
    H2 = W_pops1.shape[-1]
    OUT = W_cls.shape[-1]
    bf16 = jnp.bfloat16

    args = (
        x.astype(bf16),
        W_sel0.astype(bf16), b_sel0.reshape(1, P),
        W_pops0.astype(bf16), b_pops0,
        W_sel1.astype(bf16), b_sel1.reshape(1, P),
        W_pops1.astype(bf16), b_pops1,
        W_cls.astype(bf16), b_cls.reshape(1, OUT),
    )

    def const(shape):
        return pl.BlockSpec(shape, lambda i: (0,) * len(shape))

    return pl.pallas_call(
        _drn_body,
        grid=(T // BT,),
        in_specs=[
            pl.BlockSpec((BT, D), lambda i: (i, 0)),
            const((D, P)), const((1, P)),
            const((P, D, H1)), const((P, H1)),
            const((H1, P)), const((1, P)),
            const((P, H1, H2)), const((P, H2)),
            const((H2, OUT)), const((1, OUT)),
        ],
        out_specs=pl.BlockSpec((BT, OUT), lambda i: (i, 0)),
        out_shape=jax.ShapeDtypeStruct((T, OUT), jnp.float32),
        compiler_params=pltpu.CompilerParams(
            dimension_semantics=("parallel",),
        ),
    )(*args)


# speedup vs baseline: 1.1139x; 1.0774x over previous
import functools

import jax
import jax.numpy as jnp
from jax.experimental import pallas as pl
from jax.experimental.pallas import tpu as pltpu

BT = 512


def _mix_layer(xb, ws, bs, wp, bp):
    P = ws.shape[-1]
    logits = jnp.dot(xb, ws.astype(jnp.bfloat16),
                     preferred_element_type=jnp.float32) + bs
    m = jnp.max(logits, axis=-1, keepdims=True)
    e = jnp.exp(logits - m)
    probs = e / jnp.sum(e, axis=-1, keepdims=True)
    acc = jnp.dot(probs, bp, preferred_element_type=jnp.float32)
    for p in range(P):
        y = jnp.dot(xb, wp[p], preferred_element_type=jnp.float32)
        acc += probs[:, p : p + 1] * y
    return acc


def _drn_body(x_ref, ws0, bs0, wp0_h, bp0, ws1, bs1, wp1_h, bp1, wc_h, bc,
              o_ref, wp0_b, wp1_b, wc_b, stg, sem):
    P = wp0_b.shape[0]

    @pl.when(pl.program_id(0) == 0)
    def _load_weights():
        srcs = ([wp0_h.at[p] for p in range(P)]
                + [wp1_h.at[p] for p in range(P)] + [wc_h])
        dsts = ([wp0_b.at[p] for p in range(P)]
                + [wp1_b.at[p] for p in range(P)] + [wc_b])
        n = len(srcs)
        copies = [pltpu.make_async_copy(s, stg.at[i % 2], sem.at[i % 2])
                  for i, s in enumerate(srcs)]
        copies[0].start()
        for i in range(n):
            copies[i].wait()
            if i + 1 < n:
                copies[i + 1].start()
            dsts[i][...] = stg[i % 2].astype(jnp.bfloat16)

    xb = x_ref[...].astype(jnp.bfloat16)
    h1 = _mix_layer(xb, ws0[...], bs0[...], wp0_b, bp0[...])
    h1 = jnp.maximum(h1, 0.0).astype(jnp.bfloat16)
    h2 = _mix_layer(h1, ws1[...], bs1[...], wp1_b, bp1[...])
    h2 = jnp.maximum(h2, 0.0).astype(jnp.bfloat16)
    o_ref[...] = jnp.dot(h2, wc_b[...], preferred_element_type=jnp.float32) + bc[...]


@jax.jit
def kernel(x, W_sel0, b_sel0, W_pops0, b_pops0, W_sel1, b_sel1, W_pops1,
           b_pops1, W_cls, b_cls):
    T, D = x.shape
    P, _, H1 = W_pops0.shape
    H2 = W_pops1.shape[-1]
    OUT = W_cls.shape[-1]
    bf16 = jnp.bfloat16

    args = (
        x,
        W_sel0, b_sel0.reshape(1, P),
        W_pops0, b_pops0,
        W_sel1, b_sel1.reshape(1, P),
        W_pops1, b_pops1,
        W_cls, b_cls.reshape(1, OUT),
    )

    def const(shape):
        return pl.BlockSpec(shape, lambda i: (0,) * len(shape))

    hbm = pl.BlockSpec(memory_space=pltpu.MemorySpace.HBM)

    return pl.pallas_call(
        _drn_body,
        grid=(T // BT,),
        in_specs=[
            pl.BlockSpec((BT, D), lambda i: (i, 0)),
            const((D, P)), const((1, P)),
            hbm, const((P, H1)),
            const((H1, P)), const((1, P)),
            hbm, const((P, H2)),
            hbm, const((1, OUT)),
        ],
        out_specs=pl.BlockSpec((BT, OUT), lambda i: (i, 0)),
        out_shape=jax.ShapeDtypeStruct((T, OUT), jnp.float32),
        scratch_shapes=[
            pltpu.VMEM((P, D, H1), bf16),
            pltpu.VMEM((P, H1, H2), bf16),
            pltpu.VMEM((H2, OUT), bf16),
            pltpu.VMEM((2, D, H1), jnp.float32),
            pltpu.SemaphoreType.DMA((2,)),
        ],
        compiler_params=pltpu.CompilerParams(
            dimension_semantics=("arbitrary",),
        ),
    )(*args)
